# Initial kernel scaffold; baseline (speedup 1.0000x reference)
import functools

import jax
import jax.numpy as jnp
from jax import lax
from jax.experimental import pallas as pl
from jax.experimental.pallas import tpu as pltpu
from jax.experimental.pallas import tpu_sc as plsc

VOCAB = 100000
D = 64
B = 4096 * 200

NC = 2
NS = 16
NW = NC * NS
B_PER_W = B // NW
CHUNK = 128
NCHUNK = B_PER_W // CHUNK
G = 5
NROUND = NCHUNK // G


def _scale_body(t_ref, o_ref):
    o_ref[...] = t_ref[...] * 8.0


def _scale_table(table):
    rows_per_blk = 2000
    grid = VOCAB // rows_per_blk
    return pl.pallas_call(
        _scale_body,
        grid=(grid,),
        in_specs=[pl.BlockSpec((rows_per_blk, D), lambda i: (i, 0))],
        out_specs=pl.BlockSpec((rows_per_blk, D), lambda i: (i, 0)),
        out_shape=jax.ShapeDtypeStruct((VOCAB, D), jnp.float32),
    )(table)


def _make_sc_gather():
    mesh = plsc.VectorSubcoreMesh(core_axis_name="c", subcore_axis_name="s")

    @functools.partial(
        pl.kernel,
        mesh=mesh,
        out_type=jax.ShapeDtypeStruct((B, D), jnp.float32),
        scratch_types=[
            pltpu.VMEM((NCHUNK, CHUNK), jnp.int32),
            *[pltpu.VMEM((CHUNK, D), jnp.float32) for _ in range(2 * G)],
            pltpu.SemaphoreType.DMA,
            pltpu.SemaphoreType.DMA,
        ],
    )
    def sc_gather(table_hbm, idx_hbm, out_hbm, idx_v, *rest):
        bufs = rest[: 2 * G]
        sem_g, sem_o = rest[2 * G], rest[2 * G + 1]
        wid = lax.axis_index("s") * NC + lax.axis_index("c")
        base = wid * B_PER_W

        pltpu.sync_copy(idx_hbm.at[wid], idx_v)

        def round_body(r, gidx):
            group = bufs[gidx * G : (gidx + 1) * G]
            c0 = r * G

            @pl.when(r >= 2)
            def _():
                for b in range(G):
                    pltpu.make_async_copy(
                        group[b], out_hbm.at[pl.ds(0, CHUNK)], sem_o
                    ).wait()

            copies = []
            for b in range(G):
                copies.append(
                    pltpu.make_async_copy(
                        table_hbm.at[idx_v.at[c0 + b]], group[b], sem_g
                    )
                )
                copies[b].start()
            for b in range(G):
                copies[b].wait()

            for b in range(G):
                dst = out_hbm.at[pl.ds(base + (c0 + b) * CHUNK, CHUNK)]
                pltpu.make_async_copy(group[b], dst, sem_o).start()

        def outer(rp, carry):
            round_body(2 * rp, 0)
            round_body(2 * rp + 1, 1)
            return carry

        lax.fori_loop(0, NROUND // 2, outer, 0)

        for b in range(2 * G):
            pltpu.make_async_copy(
                bufs[b], out_hbm.at[pl.ds(0, CHUNK)], sem_o
            ).wait()

    return sc_gather


_sc_gather = _make_sc_gather()


@jax.jit
def kernel(x, table):
    scaled = _scale_table(table)
    idx = x.reshape(NW, NCHUNK, CHUNK)
    if idx.dtype != jnp.int32:
        idx = idx.astype(jnp.int32)
    out = _sc_gather(scaled, idx)
    return out.reshape(x.shape[0], x.shape[1], D)


# speedup vs baseline: 3.8668x; 3.8668x over previous
import functools

import jax
import jax.numpy as jnp
from jax import lax
from jax.experimental import pallas as pl
from jax.experimental.pallas import tpu as pltpu
from jax.experimental.pallas import tpu_sc as plsc

VOCAB = 100000
D = 64
B = 4096 * 200

NC = 2
NS = 16
NW = NC * NS
B_PER_W = B // NW
CHUNK = 128
NCHUNK = B_PER_W // CHUNK
G = 5
NROUND = NCHUNK // G


def _scale_body(t_ref, o_ref):
    o_ref[...] = t_ref[...] * 8.0


def _scale_table(table):
    rows_per_blk = 2000
    grid = VOCAB // rows_per_blk
    return pl.pallas_call(
        _scale_body,
        grid=(grid,),
        in_specs=[pl.BlockSpec((rows_per_blk, D), lambda i: (i, 0))],
        out_specs=pl.BlockSpec((rows_per_blk, D), lambda i: (i, 0)),
        out_shape=jax.ShapeDtypeStruct((VOCAB, D), jnp.float32),
    )(table)


def _make_sc_gather():
    mesh = plsc.VectorSubcoreMesh(core_axis_name="c", subcore_axis_name="s")

    @functools.partial(
        pl.kernel,
        mesh=mesh,
        out_type=jax.ShapeDtypeStruct((B, D), jnp.float32),
        scratch_types=[
            pltpu.VMEM((NCHUNK, CHUNK), jnp.int32),
            *[pltpu.VMEM((CHUNK, D), jnp.float32) for _ in range(2 * G)],
            pltpu.SemaphoreType.DMA,
            pltpu.SemaphoreType.DMA,
        ],
        compiler_params=pltpu.CompilerParams(use_tc_tiling_on_sc=False),
    )
    def sc_gather(table_hbm, idx_hbm, out_hbm, idx_v, *rest):
        bufs = rest[: 2 * G]
        sem_g, sem_o = rest[2 * G], rest[2 * G + 1]
        wid = lax.axis_index("s") * NC + lax.axis_index("c")
        base = wid * B_PER_W

        pltpu.sync_copy(idx_hbm.at[wid], idx_v)

        def round_body(r, gidx):
            group = bufs[gidx * G : (gidx + 1) * G]
            c0 = r * G

            @pl.when(r >= 2)
            def _():
                for b in range(G):
                    pltpu.make_async_copy(
                        group[b], out_hbm.at[pl.ds(0, CHUNK)], sem_o
                    ).wait()

            copies = []
            for b in range(G):
                copies.append(
                    pltpu.make_async_copy(
                        table_hbm.at[idx_v.at[c0 + b]], group[b], sem_g
                    )
                )
                copies[b].start()
            for b in range(G):
                copies[b].wait()

            for b in range(G):
                dst = out_hbm.at[pl.ds(base + (c0 + b) * CHUNK, CHUNK)]
                pltpu.make_async_copy(group[b], dst, sem_o).start()

        def outer(rp, carry):
            round_body(2 * rp, 0)
            round_body(2 * rp + 1, 1)
            return carry

        lax.fori_loop(0, NROUND // 2, outer, 0)

        for b in range(2 * G):
            pltpu.make_async_copy(
                bufs[b], out_hbm.at[pl.ds(0, CHUNK)], sem_o
            ).wait()

    return sc_gather


_sc_gather = _make_sc_gather()


@jax.jit
def kernel(x, table):
    scaled = _scale_table(table)
    idx = x.reshape(NW, NCHUNK, CHUNK)
    if idx.dtype != jnp.int32:
        idx = idx.astype(jnp.int32)
    out = _sc_gather(scaled, idx)
    return out.reshape(x.shape[0], x.shape[1], D)


# speedup vs baseline: 4.2203x; 1.0914x over previous
import functools

import jax
import jax.numpy as jnp
from jax import lax
from jax.experimental import pallas as pl
from jax.experimental.pallas import tpu as pltpu
from jax.experimental.pallas import tpu_sc as plsc

VOCAB = 100000
D = 64
BATCH = 4096
SEQ = 200

NC = 2
NS = 16
NW = NC * NS
BATCH_W = BATCH // NW
S0 = 96
S1 = 104
NCHUNK = BATCH_W * 2
G = 8
NROUND = NCHUNK // G


def _make_sc_kernel():
    mesh = plsc.VectorSubcoreMesh(core_axis_name="c", subcore_axis_name="s")

    @functools.partial(
        pl.kernel,
        mesh=mesh,
        out_type=jax.ShapeDtypeStruct((BATCH, SEQ, D), jnp.float32),
        scratch_types=[
            pltpu.VMEM((BATCH_W, S0), jnp.int32),
            pltpu.VMEM((BATCH_W, S1), jnp.int32),
            *[
                pltpu.VMEM((S0 if k % 2 == 0 else S1, D), jnp.float32)
                for k in range(2 * G)
            ],
            pltpu.SemaphoreType.DMA,
            pltpu.SemaphoreType.DMA,
            pltpu.SemaphoreType.DMA,
            pltpu.SemaphoreType.DMA,
        ],
        compiler_params=pltpu.CompilerParams(use_tc_tiling_on_sc=False),
    )
    def sc_embed(table_hbm, x_hbm, out_hbm, idx_a, idx_b, *rest):
        bufs = rest[: 2 * G]
        sem_g = rest[2 * G : 2 * G + 2]
        sem_o = rest[2 * G + 2 : 2 * G + 4]
        wid = lax.axis_index("s") * NC + lax.axis_index("c")
        b0 = wid * BATCH_W

        pltpu.sync_copy(x_hbm.at[pl.ds(b0, BATCH_W), pl.ds(0, S0)], idx_a)
        pltpu.sync_copy(x_hbm.at[pl.ds(b0, BATCH_W), pl.ds(S0, S1)], idx_b)

        def chunk_refs(r, k):
            row = r * (G // 2) + k // 2
            if k % 2 == 0:
                idx = idx_a.at[row]
                dst = out_hbm.at[b0 + row, pl.ds(0, S0)]
            else:
                idx = idx_b.at[row]
                dst = out_hbm.at[b0 + row, pl.ds(S0, S1)]
            return idx, dst

        def fire_gathers(r, gidx):
            group = bufs[gidx * G : (gidx + 1) * G]
            for k in range(G):
                idx, _ = chunk_refs(r, k)
                pltpu.make_async_copy(
                    table_hbm.at[idx], group[k], sem_g[gidx]
                ).start()

        def round_body(r, gidx):
            group = bufs[gidx * G : (gidx + 1) * G]
            other = bufs[(1 - gidx) * G : (2 - gidx) * G]

            @pl.when(r >= 1)
            def _():
                for k in range(G):
                    n = S0 if k % 2 == 0 else S1
                    pltpu.make_async_copy(
                        other[k], out_hbm.at[0, pl.ds(0, n)], sem_o[1 - gidx]
                    ).wait()

            @pl.when(r + 1 < NROUND)
            def _():
                fire_gathers(r + 1, 1 - gidx)

            for k in range(G):
                idx, _ = chunk_refs(r, k)
                pltpu.make_async_copy(
                    table_hbm.at[idx], group[k], sem_g[gidx]
                ).wait()

            for k in range(G):
                buf = group[k]
                n = S0 if k % 2 == 0 else S1

                def mul_body(i, carry, buf=buf):
                    for dj in range(4):
                        for t in range(4):
                            j = i * 4 + dj
                            sl = pl.ds(t * 16, 16)
                            buf[j, sl] = buf[j, sl] * 8.0
                    return carry

                lax.fori_loop(0, n // 4, mul_body, 0)

            for k in range(G):
                _, dst = chunk_refs(r, k)
                pltpu.make_async_copy(group[k], dst, sem_o[gidx]).start()

        fire_gathers(0, 0)

        def outer(rp, carry):
            round_body(2 * rp, 0)
            round_body(2 * rp + 1, 1)
            return carry

        lax.fori_loop(0, NROUND // 2, outer, 0)

        for k in range(G):
            n = S0 if k % 2 == 0 else S1
            pltpu.make_async_copy(
                bufs[G + k], out_hbm.at[0, pl.ds(0, n)], sem_o[1]
            ).wait()

    return sc_embed


_sc_embed = _make_sc_kernel()


@jax.jit
def kernel(x, table):
    if x.dtype != jnp.int32:
        x = x.astype(jnp.int32)
    return _sc_embed(table, x)
